# baseline (speedup 1.0000x reference)
import functools

import jax
import jax.numpy as jnp
from jax import lax
from jax.experimental import pallas as pl
from jax.experimental.pallas import tpu as pltpu
from jax.experimental.pallas import tpu_sc as plsc

_TWO_PI = 6.283185307179586
_PI = 3.141592653589793

_SPOLY = (-9.99984587e-01, 1.66632582e-01, -8.31238293e-03, 1.93161822e-04,
          -2.17321007e-06)
_CPOLY = (-9.99999443e-01, 4.99995580e-01, -4.16610316e-02, 1.38627433e-03,
          -2.42531378e-05, 2.21936942e-07)


def _poly(z, coeffs):
    acc = jnp.full((16,), coeffs[-1], dtype=jnp.float32)
    for c in reversed(coeffs[:-1]):
        acc = acc * z + c
    return acc


def _make_sc_kernel(n_tokens: int, d: int, chunk: int):
    info = plsc.get_sparse_core_info()
    nc, ns = info.num_cores, info.num_subcores
    nw = nc * ns
    assert n_tokens % (nw * chunk) == 0
    per_w = n_tokens // nw
    n_chunks = per_w // chunk
    mesh = plsc.VectorSubcoreMesh(core_axis_name="c", subcore_axis_name="s")

    @functools.partial(
        pl.kernel,
        out_type=jax.ShapeDtypeStruct((n_tokens * 2 * d,), jnp.float32),
        mesh=mesh,
        compiler_params=pltpu.CompilerParams(needs_layout_passes=False,
                                             use_tc_tiling_on_sc=False),
        scratch_types=[
            pltpu.VMEM((chunk,), jnp.int32),
            pltpu.VMEM((chunk, d), jnp.float32),
            pltpu.VMEM((chunk, d), jnp.float32),
            pltpu.VMEM((chunk * 2 * d,), jnp.float32),
            pltpu.SemaphoreType.DMA,
        ],
    )
    def torus_kernel(idx_hbm, rho_hbm, theta_hbm, out_hbm, idx_v, rows_r,
                     rows_t, out_v, sem):
        wid = lax.axis_index("s") * nc + lax.axis_index("c")
        base = wid * per_w
        iot2 = 2 * lax.iota(jnp.int32, 16)

        def chunk_body(g, _):
            tok0 = base + g * chunk
            pltpu.sync_copy(idx_hbm.at[pl.ds(tok0, chunk)], idx_v)
            cp_r = pltpu.async_copy(rho_hbm.at[idx_v], rows_r, sem)
            cp_t = pltpu.async_copy(theta_hbm.at[idx_v], rows_t, sem)
            cp_r.wait()
            cp_t.wait()

            def tok_body(j, _):
                for h in range(d // 16):
                    r = rows_r[j, pl.ds(h * 16, 16)]
                    t = rows_t[j, pl.ds(h * 16, 16)]
                    w = lax.rem(t, jnp.float32(_TWO_PI))
                    w = jnp.where(w < 0, w + jnp.float32(_TWO_PI), w)
                    x = w - jnp.float32(_PI)
                    z = x * x
                    rc = r * _poly(z, _CPOLY)
                    rs = (r * x) * _poly(z, _SPOLY)
                    off = j * (2 * d) + h * 32 + iot2
                    plsc.store_scatter(out_v, [off], rc)
                    plsc.store_scatter(out_v, [off + 1], rs)
                return ()

            lax.fori_loop(0, chunk, tok_body, (), unroll=2)
            pltpu.sync_copy(out_v, out_hbm.at[pl.ds(tok0 * 2 * d,
                                                    chunk * 2 * d)])
            return ()

        lax.fori_loop(0, n_chunks, chunk_body, ())

    return torus_kernel


def kernel(idx, rho, theta):
    b, t = idx.shape
    n = b * t
    d = rho.shape[1]
    idx_flat = idx.reshape(n)
    sc = _make_sc_kernel(n, d, chunk=128)
    out_flat = sc(idx_flat, rho, theta)
    return out_flat.reshape(b, t, 2 * d)
